# baseline (speedup 1.0000x reference)
import functools

import jax
import jax.numpy as jnp
from jax import lax
from jax.experimental import pallas as pl
from jax.experimental.pallas import tpu as pltpu
from jax.experimental.pallas import tpu_sc as plsc

_N = 10000
_NP = 10240
_D = 128
_E = 320000
_NW = 32
_CH = 128
_NCH = 80
_SLOW_C = 0
_NCHS = 48
_NCHF = 112
_TCH = 16 * (_NCHS + _NCHF)
_EP = _TCH * _CH
_RPT = _NP // 16
_RB = 1024


def _sc_mesh():
    return plsc.VectorSubcoreMesh(core_axis_name="c", subcore_axis_name="s")


def _deg_body(pk_hbm, out_hbm, idx_v, hist_v):
    c = lax.axis_index("c")
    s = lax.axis_index("s")
    w = c * 16 + s
    zero16 = jnp.zeros((16,), jnp.float32)
    ones16 = jnp.ones((16,), jnp.float32)

    def zbody(i, carry):
        hist_v[pl.ds(pl.multiple_of(i * 16, 16), 16)] = zero16
        return carry

    lax.fori_loop(0, _NP // 16, zbody, 0)

    pltpu.sync_copy(pk_hbm.at[pl.ds(w * _NCH, _NCH)], idx_v)

    def body(j, carry):
        for k in range(8):
            idx = lax.shift_right_logical(idx_v[j, pl.ds(k * 16, 16)], 14)
            plsc.addupdate_scatter(hist_v, [idx], ones16)
        return carry

    lax.fori_loop(0, _NCH, body, 0)
    pltpu.sync_copy(hist_v, out_hbm.at[w])


@jax.jit
def _degrees(edge_pk):
    return pl.kernel(
        _deg_body,
        out_type=jax.ShapeDtypeStruct((_NW, _NP), jnp.float32),
        mesh=_sc_mesh(),
        scratch_types=[
            pltpu.VMEM((_NCH, _CH), jnp.int32),
            pltpu.VMEM((_NP,), jnp.float32),
        ],
        compiler_params=pltpu.CompilerParams(needs_layout_passes=False),
    )(edge_pk)


def _agg_body(hs_hbm, pk_hbm, out_hbm, acc, rows0, rows1, pkv,
              srcb0, srcb1, dstb0, dstb1, gsem0, gsem1):
    c = lax.axis_index("c")
    s = lax.axis_index("s")

    zero16 = jnp.zeros((16,), jnp.float32)

    def zrow(r, carry):
        for k in range(8):
            rows0[r, pl.ds(k * 16, 16)] = zero16
        return carry

    lax.fori_loop(0, _CH, zrow, 0)
    rbase = pl.multiple_of(s * _RPT, 8)

    def zbody(i, carry):
        pltpu.sync_copy(
            rows0, acc.at[pl.ds(pl.multiple_of(s * _RPT + i * _CH, 8), _CH)])
        return carry

    lax.fori_loop(0, _RPT // _CH, zbody, 0)

    my_nch = jnp.where(c == _SLOW_C, _NCHS, _NCHF)
    my_base = jnp.where(c == _SLOW_C, s * _NCHS, 16 * _NCHS + s * _NCHF)

    pltpu.sync_copy(pk_hbm.at[pl.ds(my_base, _NCHF)], pkv)

    def unpack(j, srcb, dstb):
        for k in range(8):
            v = pkv[j, pl.ds(k * 16, 16)]
            srcb[pl.ds(k * 16, 16)] = lax.bitwise_and(v, 0x3FFF)
            dstb[pl.ds(k * 16, 16)] = lax.shift_right_logical(v, 14)

    plsc.subcore_barrier()

    unpack(0, srcb0, dstb0)
    pltpu.async_copy(hs_hbm.at[srcb0], rows0, gsem0)

    def body(j2, carry):
        j = j2 * 2
        unpack(j + 1, srcb1, dstb1)
        pltpu.async_copy(hs_hbm.at[srcb1], rows1, gsem1)
        pltpu.make_async_copy(hs_hbm.at[srcb0], rows0, gsem0).wait()
        pltpu.sync_copy(rows0, acc.at[dstb0], add=True)
        unpack(j + 2, srcb0, dstb0)
        pltpu.async_copy(hs_hbm.at[srcb0], rows0, gsem0)
        pltpu.make_async_copy(hs_hbm.at[srcb1], rows1, gsem1).wait()
        pltpu.sync_copy(rows1, acc.at[dstb1], add=True)
        return carry

    lax.fori_loop(0, (my_nch - 2) // 2, body, 0)
    pltpu.make_async_copy(hs_hbm.at[srcb0], rows0, gsem0).wait()
    pltpu.sync_copy(rows0, acc.at[dstb0], add=True)
    unpack(my_nch - 1, srcb1, dstb1)
    pltpu.async_copy(hs_hbm.at[srcb1], rows1, gsem1)
    pltpu.make_async_copy(hs_hbm.at[srcb1], rows1, gsem1).wait()
    pltpu.sync_copy(rows1, acc.at[dstb1], add=True)

    plsc.subcore_barrier()
    pltpu.sync_copy(acc.at[pl.ds(rbase, _RPT)],
                    out_hbm.at[c, pl.ds(rbase, _RPT)])


@jax.jit
def _aggregate(hs_pad, edge_pk):
    return pl.kernel(
        _agg_body,
        out_type=jax.ShapeDtypeStruct((2, _NP, _D), jnp.float32),
        mesh=_sc_mesh(),
        scratch_types=[
            pltpu.VMEM_SHARED((_NP, _D), jnp.float32),
            pltpu.VMEM((_CH, _D), jnp.float32),
            pltpu.VMEM((_CH, _D), jnp.float32),
            pltpu.VMEM((_NCHF, _CH), jnp.int32),
            pltpu.VMEM((_CH,), jnp.int32),
            pltpu.VMEM((_CH,), jnp.int32),
            pltpu.VMEM((_CH,), jnp.int32),
            pltpu.VMEM((_CH,), jnp.int32),
            pltpu.SemaphoreType.DMA,
            pltpu.SemaphoreType.DMA,
        ],
        compiler_params=pltpu.CompilerParams(needs_layout_passes=False),
    )(hs_pad, edge_pk)


def _hs_body(degp_ref, x_ref, w_ref, b_ref, hs_ref, dis_ref):
    deg = jnp.sum(degp_ref[...], axis=0, keepdims=True) + 1.0
    dis = lax.rsqrt(deg).T
    h = jnp.maximum(
        jnp.dot(x_ref[...], w_ref[...], preferred_element_type=jnp.float32)
        + b_ref[...], 0.0)
    hs_ref[...] = h * dis
    dis_ref[...] = dis


@jax.jit
def _hidden_scaled(deg_parts, x_pad, fc_Wt, fc_b2):
    return pl.pallas_call(
        _hs_body,
        grid=(_NP // _RB,),
        in_specs=[
            pl.BlockSpec((_NW, _RB), lambda i: (0, i)),
            pl.BlockSpec((_RB, _D), lambda i: (i, 0)),
            pl.BlockSpec((_D, _D), lambda i: (0, 0)),
            pl.BlockSpec((1, _D), lambda i: (0, 0)),
        ],
        out_specs=[
            pl.BlockSpec((_RB, _D), lambda i: (i, 0)),
            pl.BlockSpec((_RB, 1), lambda i: (i, 0)),
        ],
        out_shape=[
            jax.ShapeDtypeStruct((_NP, _D), jnp.float32),
            jax.ShapeDtypeStruct((_NP, 1), jnp.float32),
        ],
    )(deg_parts, x_pad, fc_Wt, fc_b2)


def _proj_body(dis_ref, sp_ref, hs_ref, wmu_ref, wls_ref, bmu_ref, bls_ref,
               mu_ref, ls_ref):
    ssum = sp_ref[0] + sp_ref[1]
    agg = dis_ref[...] * (ssum + hs_ref[...])
    mu_ref[...] = jnp.dot(agg, wmu_ref[...],
                          preferred_element_type=jnp.float32) + bmu_ref[...]
    ls_ref[...] = jnp.dot(agg, wls_ref[...],
                          preferred_element_type=jnp.float32) + bls_ref[...]


@jax.jit
def _project(dis, s_parts, hs, wmu_t, wls_t, bmu, bls):
    rb = 1000
    return pl.pallas_call(
        _proj_body,
        grid=(_N // rb,),
        in_specs=[
            pl.BlockSpec((rb, 1), lambda i: (i, 0)),
            pl.BlockSpec((2, rb, _D), lambda i: (0, i, 0)),
            pl.BlockSpec((rb, _D), lambda i: (i, 0)),
            pl.BlockSpec((_D, 228), lambda i: (0, 0)),
            pl.BlockSpec((_D, 228), lambda i: (0, 0)),
            pl.BlockSpec((1, 228), lambda i: (0, 0)),
            pl.BlockSpec((1, 228), lambda i: (0, 0)),
        ],
        out_specs=[
            pl.BlockSpec((rb, 228), lambda i: (i, 0)),
            pl.BlockSpec((rb, 228), lambda i: (i, 0)),
        ],
        out_shape=[
            jax.ShapeDtypeStruct((_N, 228), jnp.float32),
            jax.ShapeDtypeStruct((_N, 228), jnp.float32),
        ],
    )(dis, s_parts, hs, wmu_t, wls_t, bmu, bls)


def kernel(x, edge_index, fc_W, fc_b, conv_mu_W, conv_mu_b, conv_logstd_W,
           conv_logstd_b, addon_mu_W, addon_mu_b, addon_logstd_W,
           addon_logstd_b):
    edge_pad = jnp.pad(edge_index.astype(jnp.int32), ((0, 0), (0, _EP - _E)),
                       constant_values=_NP - 1)
    edge_pk = jnp.bitwise_or(
        edge_pad[0], jnp.left_shift(edge_pad[1], 14)).reshape(_TCH, _CH)
    x_pad = jnp.pad(x, ((0, _NP - _N), (0, 0)))

    deg_parts = _degrees(edge_pk)
    hs, dis = _hidden_scaled(deg_parts, x_pad, fc_W.T, fc_b[None, :])
    s_parts = _aggregate(hs, edge_pk)

    wmu_t = jnp.concatenate([conv_mu_W, addon_mu_W], 0).T
    wls_t = jnp.concatenate([conv_logstd_W, addon_logstd_W], 0).T
    bmu = jnp.concatenate([conv_mu_b, addon_mu_b])[None, :]
    bls = jnp.concatenate([conv_logstd_b, addon_logstd_b])[None, :]

    return _project(dis, s_parts, hs, wmu_t, wls_t, bmu, bls)


# speedup vs baseline: 1.0041x; 1.0041x over previous
import functools

import jax
import jax.numpy as jnp
from jax import lax
from jax.experimental import pallas as pl
from jax.experimental.pallas import tpu as pltpu
from jax.experimental.pallas import tpu_sc as plsc

_N = 10000
_NP = 10240
_D = 128
_E = 320000
_NW = 32
_CH = 128
_NCH = 80
_SLOW_C = 1
_NCHS = 48
_NCHF = 112
_TCH = 16 * (_NCHS + _NCHF)
_EP = _TCH * _CH
_RPT = _NP // 16
_RB = 1024


def _sc_mesh():
    return plsc.VectorSubcoreMesh(core_axis_name="c", subcore_axis_name="s")


def _deg_body(pk_hbm, out_hbm, idx_v, hist_v):
    c = lax.axis_index("c")
    s = lax.axis_index("s")
    w = c * 16 + s
    zero16 = jnp.zeros((16,), jnp.float32)
    ones16 = jnp.ones((16,), jnp.float32)

    def zbody(i, carry):
        hist_v[pl.ds(pl.multiple_of(i * 16, 16), 16)] = zero16
        return carry

    lax.fori_loop(0, _NP // 16, zbody, 0)

    pltpu.sync_copy(pk_hbm.at[pl.ds(w * _NCH, _NCH)], idx_v)

    def body(j, carry):
        for k in range(8):
            idx = lax.shift_right_logical(idx_v[j, pl.ds(k * 16, 16)], 14)
            plsc.addupdate_scatter(hist_v, [idx], ones16)
        return carry

    lax.fori_loop(0, _NCH, body, 0)
    pltpu.sync_copy(hist_v, out_hbm.at[w])


@jax.jit
def _degrees(edge_pk):
    return pl.kernel(
        _deg_body,
        out_type=jax.ShapeDtypeStruct((_NW, _NP), jnp.float32),
        mesh=_sc_mesh(),
        scratch_types=[
            pltpu.VMEM((_NCH, _CH), jnp.int32),
            pltpu.VMEM((_NP,), jnp.float32),
        ],
        compiler_params=pltpu.CompilerParams(needs_layout_passes=False),
    )(edge_pk)


def _agg_body(hs_hbm, pk_hbm, out_hbm, acc, rows0, rows1, pkv,
              srcb0, srcb1, dstb0, dstb1, gsem0, gsem1):
    c = lax.axis_index("c")
    s = lax.axis_index("s")

    zero16 = jnp.zeros((16,), jnp.float32)

    def zrow(r, carry):
        for k in range(8):
            rows0[r, pl.ds(k * 16, 16)] = zero16
        return carry

    lax.fori_loop(0, _CH, zrow, 0)
    rbase = pl.multiple_of(s * _RPT, 8)

    def zbody(i, carry):
        pltpu.sync_copy(
            rows0, acc.at[pl.ds(pl.multiple_of(s * _RPT + i * _CH, 8), _CH)])
        return carry

    lax.fori_loop(0, _RPT // _CH, zbody, 0)

    my_nch = jnp.where(c == _SLOW_C, _NCHS, _NCHF)
    my_base = jnp.where(c == _SLOW_C, s * _NCHS, 16 * _NCHS + s * _NCHF)

    pltpu.sync_copy(pk_hbm.at[pl.ds(my_base, _NCHF)], pkv)

    def unpack(j, srcb, dstb):
        for k in range(8):
            v = pkv[j, pl.ds(k * 16, 16)]
            srcb[pl.ds(k * 16, 16)] = lax.bitwise_and(v, 0x3FFF)
            dstb[pl.ds(k * 16, 16)] = lax.shift_right_logical(v, 14)

    plsc.subcore_barrier()

    unpack(0, srcb0, dstb0)
    pltpu.async_copy(hs_hbm.at[srcb0], rows0, gsem0)

    def body(j2, carry):
        j = j2 * 2
        unpack(j + 1, srcb1, dstb1)
        pltpu.async_copy(hs_hbm.at[srcb1], rows1, gsem1)
        pltpu.make_async_copy(hs_hbm.at[srcb0], rows0, gsem0).wait()
        pltpu.sync_copy(rows0, acc.at[dstb0], add=True)
        unpack(j + 2, srcb0, dstb0)
        pltpu.async_copy(hs_hbm.at[srcb0], rows0, gsem0)
        pltpu.make_async_copy(hs_hbm.at[srcb1], rows1, gsem1).wait()
        pltpu.sync_copy(rows1, acc.at[dstb1], add=True)
        return carry

    lax.fori_loop(0, (my_nch - 2) // 2, body, 0)
    pltpu.make_async_copy(hs_hbm.at[srcb0], rows0, gsem0).wait()
    pltpu.sync_copy(rows0, acc.at[dstb0], add=True)
    unpack(my_nch - 1, srcb1, dstb1)
    pltpu.async_copy(hs_hbm.at[srcb1], rows1, gsem1)
    pltpu.make_async_copy(hs_hbm.at[srcb1], rows1, gsem1).wait()
    pltpu.sync_copy(rows1, acc.at[dstb1], add=True)

    plsc.subcore_barrier()
    pltpu.sync_copy(acc.at[pl.ds(rbase, _RPT)],
                    out_hbm.at[c, pl.ds(rbase, _RPT)])


@jax.jit
def _aggregate(hs_pad, edge_pk):
    return pl.kernel(
        _agg_body,
        out_type=jax.ShapeDtypeStruct((2, _NP, _D), jnp.float32),
        mesh=_sc_mesh(),
        scratch_types=[
            pltpu.VMEM_SHARED((_NP, _D), jnp.float32),
            pltpu.VMEM((_CH, _D), jnp.float32),
            pltpu.VMEM((_CH, _D), jnp.float32),
            pltpu.VMEM((_NCHF, _CH), jnp.int32),
            pltpu.VMEM((_CH,), jnp.int32),
            pltpu.VMEM((_CH,), jnp.int32),
            pltpu.VMEM((_CH,), jnp.int32),
            pltpu.VMEM((_CH,), jnp.int32),
            pltpu.SemaphoreType.DMA,
            pltpu.SemaphoreType.DMA,
        ],
        compiler_params=pltpu.CompilerParams(needs_layout_passes=False),
    )(hs_pad, edge_pk)


def _hs_body(degp_ref, x_ref, w_ref, b_ref, hs_ref, dis_ref):
    deg = jnp.sum(degp_ref[...], axis=0, keepdims=True) + 1.0
    dis = lax.rsqrt(deg).T
    h = jnp.maximum(
        jnp.dot(x_ref[...], w_ref[...], preferred_element_type=jnp.float32)
        + b_ref[...], 0.0)
    hs_ref[...] = h * dis
    dis_ref[...] = dis


@jax.jit
def _hidden_scaled(deg_parts, x_pad, fc_Wt, fc_b2):
    return pl.pallas_call(
        _hs_body,
        grid=(_NP // _RB,),
        in_specs=[
            pl.BlockSpec((_NW, _RB), lambda i: (0, i)),
            pl.BlockSpec((_RB, _D), lambda i: (i, 0)),
            pl.BlockSpec((_D, _D), lambda i: (0, 0)),
            pl.BlockSpec((1, _D), lambda i: (0, 0)),
        ],
        out_specs=[
            pl.BlockSpec((_RB, _D), lambda i: (i, 0)),
            pl.BlockSpec((_RB, 1), lambda i: (i, 0)),
        ],
        out_shape=[
            jax.ShapeDtypeStruct((_NP, _D), jnp.float32),
            jax.ShapeDtypeStruct((_NP, 1), jnp.float32),
        ],
    )(deg_parts, x_pad, fc_Wt, fc_b2)


def _proj_body(dis_ref, sp_ref, hs_ref, wmu_ref, wls_ref, bmu_ref, bls_ref,
               mu_ref, ls_ref):
    ssum = sp_ref[0] + sp_ref[1]
    agg = dis_ref[...] * (ssum + hs_ref[...])
    mu_ref[...] = jnp.dot(agg, wmu_ref[...],
                          preferred_element_type=jnp.float32) + bmu_ref[...]
    ls_ref[...] = jnp.dot(agg, wls_ref[...],
                          preferred_element_type=jnp.float32) + bls_ref[...]


@jax.jit
def _project(dis, s_parts, hs, wmu_t, wls_t, bmu, bls):
    rb = 1000
    return pl.pallas_call(
        _proj_body,
        grid=(_N // rb,),
        in_specs=[
            pl.BlockSpec((rb, 1), lambda i: (i, 0)),
            pl.BlockSpec((2, rb, _D), lambda i: (0, i, 0)),
            pl.BlockSpec((rb, _D), lambda i: (i, 0)),
            pl.BlockSpec((_D, 228), lambda i: (0, 0)),
            pl.BlockSpec((_D, 228), lambda i: (0, 0)),
            pl.BlockSpec((1, 228), lambda i: (0, 0)),
            pl.BlockSpec((1, 228), lambda i: (0, 0)),
        ],
        out_specs=[
            pl.BlockSpec((rb, 228), lambda i: (i, 0)),
            pl.BlockSpec((rb, 228), lambda i: (i, 0)),
        ],
        out_shape=[
            jax.ShapeDtypeStruct((_N, 228), jnp.float32),
            jax.ShapeDtypeStruct((_N, 228), jnp.float32),
        ],
    )(dis, s_parts, hs, wmu_t, wls_t, bmu, bls)


def kernel(x, edge_index, fc_W, fc_b, conv_mu_W, conv_mu_b, conv_logstd_W,
           conv_logstd_b, addon_mu_W, addon_mu_b, addon_logstd_W,
           addon_logstd_b):
    edge_pad = jnp.pad(edge_index.astype(jnp.int32), ((0, 0), (0, _EP - _E)),
                       constant_values=_NP - 1)
    edge_pk = jnp.bitwise_or(
        edge_pad[0], jnp.left_shift(edge_pad[1], 14)).reshape(_TCH, _CH)
    x_pad = jnp.pad(x, ((0, _NP - _N), (0, 0)))

    deg_parts = _degrees(edge_pk)
    hs, dis = _hidden_scaled(deg_parts, x_pad, fc_W.T, fc_b[None, :])
    s_parts = _aggregate(hs, edge_pk)

    wmu_t = jnp.concatenate([conv_mu_W, addon_mu_W], 0).T
    wls_t = jnp.concatenate([conv_logstd_W, addon_logstd_W], 0).T
    bmu = jnp.concatenate([conv_mu_b, addon_mu_b])[None, :]
    bls = jnp.concatenate([conv_logstd_b, addon_logstd_b])[None, :]

    return _project(dis, s_parts, hs, wmu_t, wls_t, bmu, bls)


# speedup vs baseline: 1.0257x; 1.0215x over previous
import functools

import jax
import jax.numpy as jnp
from jax import lax
from jax.experimental import pallas as pl
from jax.experimental.pallas import tpu as pltpu
from jax.experimental.pallas import tpu_sc as plsc

_N = 10000
_NP = 10240
_D = 128
_E = 320000
_NW = 32
_CH = 128
_NCH = 80
_TCH = _NW * _NCH
_EP = _TCH * _CH
_RPT = _NP // 16
_RB = 1024


def _sc_mesh():
    return plsc.VectorSubcoreMesh(core_axis_name="c", subcore_axis_name="s")


def _deg_body(pk_hbm, out_hbm, idx_v, hist_v):
    c = lax.axis_index("c")
    s = lax.axis_index("s")
    w = c * 16 + s
    zero16 = jnp.zeros((16,), jnp.float32)
    ones16 = jnp.ones((16,), jnp.float32)

    def zbody(i, carry):
        hist_v[pl.ds(pl.multiple_of(i * 16, 16), 16)] = zero16
        return carry

    lax.fori_loop(0, _NP // 16, zbody, 0)

    pltpu.sync_copy(pk_hbm.at[pl.ds(w * _NCH, _NCH)], idx_v)

    def body(j, carry):
        for k in range(8):
            idx = lax.shift_right_logical(idx_v[j, pl.ds(k * 16, 16)], 14)
            plsc.addupdate_scatter(hist_v, [idx], ones16)
        return carry

    lax.fori_loop(0, _NCH, body, 0)
    pltpu.sync_copy(hist_v, out_hbm.at[w])


@jax.jit
def _degrees(edge_pk):
    return pl.kernel(
        _deg_body,
        out_type=jax.ShapeDtypeStruct((_NW, _NP), jnp.float32),
        mesh=_sc_mesh(),
        scratch_types=[
            pltpu.VMEM((_NCH, _CH), jnp.int32),
            pltpu.VMEM((_NP,), jnp.float32),
        ],
        compiler_params=pltpu.CompilerParams(needs_layout_passes=False),
    )(edge_pk)


def _agg_body(hs_hbm, pk_hbm, out_hbm, acc, rows0, rows1, pkv,
              srcb0, srcb1, dstb0, dstb1, gsem0, gsem1):
    c = lax.axis_index("c")
    s = lax.axis_index("s")

    zero16 = jnp.zeros((16,), jnp.float32)

    def zrow(r, carry):
        for k in range(8):
            rows0[r, pl.ds(k * 16, 16)] = zero16
        return carry

    lax.fori_loop(0, _CH, zrow, 0)
    rbase = pl.multiple_of(s * _RPT, 8)

    def zbody(i, carry):
        pltpu.sync_copy(
            rows0, acc.at[pl.ds(pl.multiple_of(s * _RPT + i * _CH, 8), _CH)])
        return carry

    lax.fori_loop(0, _RPT // _CH, zbody, 0)

    w = c * 16 + s
    pltpu.sync_copy(pk_hbm.at[pl.ds(w * _NCH, _NCH)], pkv)

    def unpack(j, srcb, dstb):
        for k in range(8):
            v = pkv[j, pl.ds(k * 16, 16)]
            srcb[pl.ds(k * 16, 16)] = lax.bitwise_and(v, 0x3FFF)
            dstb[pl.ds(k * 16, 16)] = lax.shift_right_logical(v, 14)

    plsc.subcore_barrier()

    unpack(0, srcb0, dstb0)
    pltpu.async_copy(hs_hbm.at[srcb0], rows0, gsem0)

    def body(j2, carry):
        j = j2 * 2
        unpack(j + 1, srcb1, dstb1)
        pltpu.async_copy(hs_hbm.at[srcb1], rows1, gsem1)
        pltpu.make_async_copy(hs_hbm.at[srcb0], rows0, gsem0).wait()
        pltpu.sync_copy(rows0, acc.at[dstb0], add=True)
        unpack(j + 2, srcb0, dstb0)
        pltpu.async_copy(hs_hbm.at[srcb0], rows0, gsem0)
        pltpu.make_async_copy(hs_hbm.at[srcb1], rows1, gsem1).wait()
        pltpu.sync_copy(rows1, acc.at[dstb1], add=True)
        return carry

    lax.fori_loop(0, (_NCH - 2) // 2, body, 0)
    pltpu.make_async_copy(hs_hbm.at[srcb0], rows0, gsem0).wait()
    pltpu.sync_copy(rows0, acc.at[dstb0], add=True)
    unpack(_NCH - 1, srcb1, dstb1)
    pltpu.async_copy(hs_hbm.at[srcb1], rows1, gsem1)
    pltpu.make_async_copy(hs_hbm.at[srcb1], rows1, gsem1).wait()
    pltpu.sync_copy(rows1, acc.at[dstb1], add=True)

    plsc.subcore_barrier()
    pltpu.sync_copy(acc.at[pl.ds(rbase, _RPT)],
                    out_hbm.at[c, pl.ds(rbase, _RPT)])


@jax.jit
def _aggregate(hs_pad, edge_pk):
    return pl.kernel(
        _agg_body,
        out_type=jax.ShapeDtypeStruct((2, _NP, _D), jnp.float32),
        mesh=_sc_mesh(),
        scratch_types=[
            pltpu.VMEM_SHARED((_NP, _D), jnp.float32),
            pltpu.VMEM((_CH, _D), jnp.float32),
            pltpu.VMEM((_CH, _D), jnp.float32),
            pltpu.VMEM((_NCH, _CH), jnp.int32),
            pltpu.VMEM((_CH,), jnp.int32),
            pltpu.VMEM((_CH,), jnp.int32),
            pltpu.VMEM((_CH,), jnp.int32),
            pltpu.VMEM((_CH,), jnp.int32),
            pltpu.SemaphoreType.DMA,
            pltpu.SemaphoreType.DMA,
        ],
        compiler_params=pltpu.CompilerParams(needs_layout_passes=False),
    )(hs_pad, edge_pk)


def _hs_body(degp_ref, x_ref, w_ref, b_ref, hs_ref, dis_ref):
    deg = jnp.sum(degp_ref[...], axis=0, keepdims=True) + 1.0
    dis = lax.rsqrt(deg).T
    h = jnp.maximum(
        jnp.dot(x_ref[...], w_ref[...], preferred_element_type=jnp.float32)
        + b_ref[...], 0.0)
    hs_ref[...] = h * dis
    dis_ref[...] = dis


@jax.jit
def _hidden_scaled(deg_parts, x_pad, fc_Wt, fc_b2):
    return pl.pallas_call(
        _hs_body,
        grid=(_NP // _RB,),
        in_specs=[
            pl.BlockSpec((_NW, _RB), lambda i: (0, i)),
            pl.BlockSpec((_RB, _D), lambda i: (i, 0)),
            pl.BlockSpec((_D, _D), lambda i: (0, 0)),
            pl.BlockSpec((1, _D), lambda i: (0, 0)),
        ],
        out_specs=[
            pl.BlockSpec((_RB, _D), lambda i: (i, 0)),
            pl.BlockSpec((_RB, 1), lambda i: (i, 0)),
        ],
        out_shape=[
            jax.ShapeDtypeStruct((_NP, _D), jnp.float32),
            jax.ShapeDtypeStruct((_NP, 1), jnp.float32),
        ],
    )(deg_parts, x_pad, fc_Wt, fc_b2)


def _proj_body(dis_ref, sp_ref, hs_ref, wmu_ref, wls_ref, bmu_ref, bls_ref,
               mu_ref, ls_ref):
    ssum = sp_ref[0] + sp_ref[1]
    agg = dis_ref[...] * (ssum + hs_ref[...])
    mu_ref[...] = jnp.dot(agg, wmu_ref[...],
                          preferred_element_type=jnp.float32) + bmu_ref[...]
    ls_ref[...] = jnp.dot(agg, wls_ref[...],
                          preferred_element_type=jnp.float32) + bls_ref[...]


@jax.jit
def _project(dis, s_parts, hs, wmu_t, wls_t, bmu, bls):
    rb = 1000
    return pl.pallas_call(
        _proj_body,
        grid=(_N // rb,),
        in_specs=[
            pl.BlockSpec((rb, 1), lambda i: (i, 0)),
            pl.BlockSpec((2, rb, _D), lambda i: (0, i, 0)),
            pl.BlockSpec((rb, _D), lambda i: (i, 0)),
            pl.BlockSpec((_D, 228), lambda i: (0, 0)),
            pl.BlockSpec((_D, 228), lambda i: (0, 0)),
            pl.BlockSpec((1, 228), lambda i: (0, 0)),
            pl.BlockSpec((1, 228), lambda i: (0, 0)),
        ],
        out_specs=[
            pl.BlockSpec((rb, 228), lambda i: (i, 0)),
            pl.BlockSpec((rb, 228), lambda i: (i, 0)),
        ],
        out_shape=[
            jax.ShapeDtypeStruct((_N, 228), jnp.float32),
            jax.ShapeDtypeStruct((_N, 228), jnp.float32),
        ],
    )(dis, s_parts, hs, wmu_t, wls_t, bmu, bls)


def kernel(x, edge_index, fc_W, fc_b, conv_mu_W, conv_mu_b, conv_logstd_W,
           conv_logstd_b, addon_mu_W, addon_mu_b, addon_logstd_W,
           addon_logstd_b):
    edge_pad = jnp.pad(edge_index.astype(jnp.int32), ((0, 0), (0, _EP - _E)),
                       constant_values=_NP - 1)
    edge_pk = jnp.bitwise_or(
        edge_pad[0], jnp.left_shift(edge_pad[1], 14)).reshape(_TCH, _CH)
    x_pad = jnp.pad(x, ((0, _NP - _N), (0, 0)))

    deg_parts = _degrees(edge_pk)
    hs, dis = _hidden_scaled(deg_parts, x_pad, fc_W.T, fc_b[None, :])
    s_parts = _aggregate(hs, edge_pk)

    wmu_t = jnp.concatenate([conv_mu_W, addon_mu_W], 0).T
    wls_t = jnp.concatenate([conv_logstd_W, addon_logstd_W], 0).T
    bmu = jnp.concatenate([conv_mu_b, addon_mu_b])[None, :]
    bls = jnp.concatenate([conv_logstd_b, addon_logstd_b])[None, :]

    return _project(dis, s_parts, hs, wmu_t, wls_t, bmu, bls)


# speedup vs baseline: 1.0515x; 1.0251x over previous
import functools

import jax
import jax.numpy as jnp
from jax import lax
from jax.experimental import pallas as pl
from jax.experimental.pallas import tpu as pltpu
from jax.experimental.pallas import tpu_sc as plsc

_N = 10000
_NP = 10240
_D = 128
_E = 320000
_NW = 32
_CH = 128
_NCH = 80
_SLOW_C = 0
_NCHS = 48
_NCHF = 112
_TCH = 16 * (_NCHS + _NCHF)
_EP = _TCH * _CH
_RPT = _NP // 16
_RB = 1024


def _sc_mesh():
    return plsc.VectorSubcoreMesh(core_axis_name="c", subcore_axis_name="s")


def _deg_body(pk_hbm, out_hbm, idx_v, hist_v):
    c = lax.axis_index("c")
    s = lax.axis_index("s")
    w = c * 16 + s
    zero16 = jnp.zeros((16,), jnp.float32)
    ones16 = jnp.ones((16,), jnp.float32)

    def zbody(i, carry):
        hist_v[pl.ds(pl.multiple_of(i * 16, 16), 16)] = zero16
        return carry

    lax.fori_loop(0, _NP // 16, zbody, 0)

    pltpu.sync_copy(pk_hbm.at[pl.ds(w * _NCH, _NCH)], idx_v)

    def body(j, carry):
        for k in range(8):
            idx = lax.shift_right_logical(idx_v[j, pl.ds(k * 16, 16)], 14)
            plsc.addupdate_scatter(hist_v, [idx], ones16)
        return carry

    lax.fori_loop(0, _NCH, body, 0)
    pltpu.sync_copy(hist_v, out_hbm.at[w])


@jax.jit
def _degrees(edge_pk):
    return pl.kernel(
        _deg_body,
        out_type=jax.ShapeDtypeStruct((_NW, _NP), jnp.float32),
        mesh=_sc_mesh(),
        scratch_types=[
            pltpu.VMEM((_NCH, _CH), jnp.int32),
            pltpu.VMEM((_NP,), jnp.float32),
        ],
        compiler_params=pltpu.CompilerParams(needs_layout_passes=False),
    )(edge_pk)


def _agg_body(hs_hbm, pk_hbm, out_hbm, acc, rows0, rows1, pkv,
              srcb0, srcb1, dstb0, dstb1, gsem0, gsem1):
    c = lax.axis_index("c")
    s = lax.axis_index("s")

    zero16 = jnp.zeros((16,), jnp.float32)

    def zrow(r, carry):
        for k in range(8):
            rows0[r, pl.ds(k * 16, 16)] = zero16
        return carry

    lax.fori_loop(0, _CH, zrow, 0)
    rbase = pl.multiple_of(s * _RPT, 8)

    def zbody(i, carry):
        pltpu.sync_copy(
            rows0, acc.at[pl.ds(pl.multiple_of(s * _RPT + i * _CH, 8), _CH)])
        return carry

    lax.fori_loop(0, _RPT // _CH, zbody, 0)

    my_nch = jnp.where(c == _SLOW_C, _NCHS, _NCHF)
    my_base = jnp.where(c == _SLOW_C, s * _NCHS, 16 * _NCHS + s * _NCHF)

    pltpu.sync_copy(pk_hbm.at[pl.ds(my_base, _NCHF)], pkv)

    def unpack(j, srcb, dstb):
        for k in range(8):
            v = pkv[j, pl.ds(k * 16, 16)]
            srcb[pl.ds(k * 16, 16)] = lax.bitwise_and(v, 0x3FFF)
            dstb[pl.ds(k * 16, 16)] = lax.shift_right_logical(v, 14)

    plsc.subcore_barrier()

    unpack(0, srcb0, dstb0)
    pltpu.async_copy(hs_hbm.at[srcb0], rows0, gsem0)

    def body(j2, carry):
        j = j2 * 2
        unpack(j + 1, srcb1, dstb1)
        pltpu.async_copy(hs_hbm.at[srcb1], rows1, gsem1)
        pltpu.make_async_copy(hs_hbm.at[srcb0], rows0, gsem0).wait()
        pltpu.sync_copy(rows0, acc.at[dstb0], add=True)
        unpack(j + 2, srcb0, dstb0)
        pltpu.async_copy(hs_hbm.at[srcb0], rows0, gsem0)
        pltpu.make_async_copy(hs_hbm.at[srcb1], rows1, gsem1).wait()
        pltpu.sync_copy(rows1, acc.at[dstb1], add=True)
        return carry

    lax.fori_loop(0, (my_nch - 2) // 2, body, 0)
    pltpu.make_async_copy(hs_hbm.at[srcb0], rows0, gsem0).wait()
    pltpu.sync_copy(rows0, acc.at[dstb0], add=True)
    unpack(my_nch - 1, srcb1, dstb1)
    pltpu.async_copy(hs_hbm.at[srcb1], rows1, gsem1)
    pltpu.make_async_copy(hs_hbm.at[srcb1], rows1, gsem1).wait()
    pltpu.sync_copy(rows1, acc.at[dstb1], add=True)

    plsc.subcore_barrier()
    pltpu.sync_copy(acc.at[pl.ds(rbase, _RPT)],
                    out_hbm.at[c, pl.ds(rbase, _RPT)])


@jax.jit
def _aggregate(hs_pad, edge_pk):
    return pl.kernel(
        _agg_body,
        out_type=jax.ShapeDtypeStruct((2, _NP, _D), jnp.float32),
        mesh=_sc_mesh(),
        scratch_types=[
            pltpu.VMEM_SHARED((_NP, _D), jnp.float32),
            pltpu.VMEM((_CH, _D), jnp.float32),
            pltpu.VMEM((_CH, _D), jnp.float32),
            pltpu.VMEM((_NCHF, _CH), jnp.int32),
            pltpu.VMEM((_CH,), jnp.int32),
            pltpu.VMEM((_CH,), jnp.int32),
            pltpu.VMEM((_CH,), jnp.int32),
            pltpu.VMEM((_CH,), jnp.int32),
            pltpu.SemaphoreType.DMA,
            pltpu.SemaphoreType.DMA,
        ],
        compiler_params=pltpu.CompilerParams(needs_layout_passes=False),
    )(hs_pad, edge_pk)


def _hs_body(degp_ref, x_ref, w_ref, b_ref, hs_ref, dis_ref):
    deg = jnp.sum(degp_ref[...], axis=0, keepdims=True) + 1.0
    dis = lax.rsqrt(deg).T
    h = jnp.maximum(
        jnp.dot(x_ref[...], w_ref[...], preferred_element_type=jnp.float32)
        + b_ref[...], 0.0)
    hs_ref[...] = h * dis
    dis_ref[...] = dis


@jax.jit
def _hidden_scaled(deg_parts, x_pad, fc_Wt, fc_b2):
    return pl.pallas_call(
        _hs_body,
        grid=(_NP // _RB,),
        in_specs=[
            pl.BlockSpec((_NW, _RB), lambda i: (0, i)),
            pl.BlockSpec((_RB, _D), lambda i: (i, 0)),
            pl.BlockSpec((_D, _D), lambda i: (0, 0)),
            pl.BlockSpec((1, _D), lambda i: (0, 0)),
        ],
        out_specs=[
            pl.BlockSpec((_RB, _D), lambda i: (i, 0)),
            pl.BlockSpec((_RB, 1), lambda i: (i, 0)),
        ],
        out_shape=[
            jax.ShapeDtypeStruct((_NP, _D), jnp.float32),
            jax.ShapeDtypeStruct((_NP, 1), jnp.float32),
        ],
    )(deg_parts, x_pad, fc_Wt, fc_b2)


def _proj_body(dis_ref, sp_ref, hs_ref, wmu_ref, wls_ref, bmu_ref, bls_ref,
               mu_ref, ls_ref):
    ssum = sp_ref[0] + sp_ref[1]
    agg = dis_ref[...] * (ssum + hs_ref[...])
    mu_ref[...] = jnp.dot(agg, wmu_ref[...],
                          preferred_element_type=jnp.float32) + bmu_ref[...]
    ls_ref[...] = jnp.dot(agg, wls_ref[...],
                          preferred_element_type=jnp.float32) + bls_ref[...]


@jax.jit
def _project(dis, s_parts, hs, wmu_t, wls_t, bmu, bls):
    rb = 1000
    return pl.pallas_call(
        _proj_body,
        grid=(_N // rb,),
        in_specs=[
            pl.BlockSpec((rb, 1), lambda i: (i, 0)),
            pl.BlockSpec((2, rb, _D), lambda i: (0, i, 0)),
            pl.BlockSpec((rb, _D), lambda i: (i, 0)),
            pl.BlockSpec((_D, 228), lambda i: (0, 0)),
            pl.BlockSpec((_D, 228), lambda i: (0, 0)),
            pl.BlockSpec((1, 228), lambda i: (0, 0)),
            pl.BlockSpec((1, 228), lambda i: (0, 0)),
        ],
        out_specs=[
            pl.BlockSpec((rb, 228), lambda i: (i, 0)),
            pl.BlockSpec((rb, 228), lambda i: (i, 0)),
        ],
        out_shape=[
            jax.ShapeDtypeStruct((_N, 228), jnp.float32),
            jax.ShapeDtypeStruct((_N, 228), jnp.float32),
        ],
    )(dis, s_parts, hs, wmu_t, wls_t, bmu, bls)


def kernel(x, edge_index, fc_W, fc_b, conv_mu_W, conv_mu_b, conv_logstd_W,
           conv_logstd_b, addon_mu_W, addon_mu_b, addon_logstd_W,
           addon_logstd_b):
    edge_pad = jnp.pad(edge_index.astype(jnp.int32), ((0, 0), (0, _EP - _E)),
                       constant_values=_NP - 1)
    edge_pk = jnp.bitwise_or(
        edge_pad[0], jnp.left_shift(edge_pad[1], 14)).reshape(_TCH, _CH)
    x_pad = jnp.pad(x, ((0, _NP - _N), (0, 0)))

    deg_parts = _degrees(edge_pk)
    hs, dis = _hidden_scaled(deg_parts, x_pad, fc_W.T, fc_b[None, :])
    s_parts = _aggregate(hs, edge_pk)

    wmu_t = jnp.concatenate([conv_mu_W, addon_mu_W], 0).T
    wls_t = jnp.concatenate([conv_logstd_W, addon_logstd_W], 0).T
    bmu = jnp.concatenate([conv_mu_b, addon_mu_b])[None, :]
    bls = jnp.concatenate([conv_logstd_b, addon_logstd_b])[None, :]

    return _project(dis, s_parts, hs, wmu_t, wls_t, bmu, bls)
